# Initial kernel scaffold; baseline (speedup 1.0000x reference)
import jax
import jax.numpy as jnp
from jax.experimental import pallas as pl


def kernel(ego_embeddings, edge_index, edge_weight, W1, b1, W2, b2):
    raise NotImplementedError("write your pallas kernel here")


# speedup vs baseline: 4.2207x; 4.2207x over previous
import functools

import jax
import jax.numpy as jnp
from jax import lax
from jax.experimental import pallas as pl
from jax.experimental.pallas import tpu as pltpu
from jax.experimental.pallas import tpu_sc as plsc

NC = 2
NS = 16
L = 16
NW = NC * NS

CHUNK = 80


def _sc_side_partials(n_nodes: int, n_edges: int, d: int):
    assert d % L == 0
    assert n_edges % (NW * CHUNK) == 0
    e_per_w = n_edges // NW
    n_chunks = e_per_w // CHUNK
    assert n_nodes % 16 == 0
    base_rows = (n_nodes // NS) // 16 * 16
    last_rows = n_nodes - base_rows * (NS - 1)
    zrows = 16

    mesh = plsc.VectorSubcoreMesh(
        core_axis_name="c", subcore_axis_name="s", num_cores=NC, num_subcores=NS
    )

    @functools.partial(
        pl.kernel,
        out_type=jax.ShapeDtypeStruct((NC, n_nodes, d), jnp.float32),
        mesh=mesh,
        scratch_types=[
            pltpu.VMEM((CHUNK,), jnp.int32),
            pltpu.VMEM((CHUNK,), jnp.int32),
            pltpu.VMEM((CHUNK,), jnp.float32),
            pltpu.VMEM((CHUNK, d), jnp.float32),
            pltpu.VMEM((zrows, d), jnp.float32),
            pltpu.VMEM_SHARED((n_nodes, d), jnp.float32),
            pltpu.SemaphoreType.DMA,
        ],
    )
    def sc_kernel(src_hbm, dst_hbm, w_hbm, ego_hbm, out_hbm,
                  src_v, dst_v, w_v, rows_v, zbuf, acc, sem):
        cid = lax.axis_index("c")
        sid = lax.axis_index("s")
        wid = sid * NC + cid
        row_start = sid * base_rows
        n_blk = jnp.where(sid == NS - 1, last_rows // 16, base_rows // 16)

        for i in range(zrows):
            for j in range(d // L):
                zbuf[i, pl.ds(j * L, L)] = jnp.zeros((L,), jnp.float32)

        def zero_blk(i, _):
            pltpu.sync_copy(zbuf, acc.at[pl.ds(row_start + i * 16, 16)])
            return 0
        lax.fori_loop(0, n_blk, zero_blk, 0)
        plsc.subcore_barrier()

        def chunk_body(cidx, _):
            base = wid * e_per_w + cidx * CHUNK
            pltpu.sync_copy(src_hbm.at[pl.ds(base, CHUNK)], src_v)
            pltpu.sync_copy(dst_hbm.at[pl.ds(base, CHUNK)], dst_v)
            pltpu.sync_copy(w_hbm.at[pl.ds(base, CHUNK)], w_v)
            pltpu.async_copy(ego_hbm.at[src_v], rows_v, sem).wait()

            def scale_group(g, _):
                w16 = w_v[pl.ds(g * L, L)]
                for k in range(L):
                    wsplat = w16.at[jnp.full((L,), k, jnp.int32)].get(
                        mode="promise_in_bounds")
                    r = g * L + k
                    for j in range(d // L):
                        sl = pl.ds(j * L, L)
                        rows_v[r, sl] = rows_v[r, sl] * wsplat
                return 0
            lax.fori_loop(0, CHUNK // L, scale_group, 0)

            pltpu.sync_copy(rows_v, acc.at[dst_v], add=True)
            return 0
        lax.fori_loop(0, n_chunks, chunk_body, 0)
        plsc.subcore_barrier()

        def out_blk(i, _):
            pltpu.sync_copy(
                acc.at[pl.ds(row_start + i * 16, 16)],
                out_hbm.at[cid, pl.ds(row_start + i * 16, 16)],
            )
            return 0
        lax.fori_loop(0, n_blk, out_blk, 0)

    return sc_kernel


def _tc_combine(ego, p0, p1, W1, b1, W2, b2):
    n, d = ego.shape
    blk = 400
    assert n % blk == 0

    def body(ego_r, p0_r, p1_r, w1_r, b1_r, w2_r, b2_r, out_r):
        side = p0_r[...] + p1_r[...]
        e = ego_r[...]
        s = jnp.dot(e + side, w1_r[...], preferred_element_type=jnp.float32) + b1_r[...]
        t = jnp.dot(e * side, w2_r[...], preferred_element_type=jnp.float32) + b2_r[...]
        out_r[...] = jnp.where(s >= 0, s, 0.01 * s) + jnp.where(t >= 0, t, 0.01 * t)

    row_spec = pl.BlockSpec((blk, d), lambda i: (i, 0))
    full_spec = pl.BlockSpec((d, d), lambda i: (0, 0))
    vec_spec = pl.BlockSpec((1, d), lambda i: (0, 0))
    return pl.pallas_call(
        body,
        grid=(n // blk,),
        in_specs=[row_spec, row_spec, row_spec, full_spec, vec_spec, full_spec, vec_spec],
        out_specs=row_spec,
        out_shape=jax.ShapeDtypeStruct((n, d), jnp.float32),
    )(ego, p0, p1, W1, b1.reshape(1, d), W2, b2.reshape(1, d))


def kernel(ego_embeddings, edge_index, edge_weight, W1, b1, W2, b2):
    n, d = ego_embeddings.shape
    e = edge_index.shape[1]
    src = edge_index[0]
    dst = edge_index[1]
    partials = _sc_side_partials(n, e, d)(src, dst, edge_weight, ego_embeddings)
    return _tc_combine(ego_embeddings, partials[0], partials[1], W1, b1, W2, b2)
